# initial kernel scaffold (unmeasured)
import jax
import jax.numpy as jnp
from jax import lax
from jax.experimental import pallas as pl
from jax.experimental.pallas import tpu as pltpu

N_GLOBAL = 4096
EPS = 1e-5
BLK = 512


def kernel(x, gamma):
    m, n = x.shape
    nb = m // BLK
    gamma2 = gamma.reshape(1, n)

    def body(x_ref, g_ref, out_ref, cache_ref, psum_ref, peer_ref,
             send_sem, recv_sem):
        p = pl.program_id(0)
        b = pl.program_id(1)
        my_x = lax.axis_index("x")
        my_y = lax.axis_index("y")

        @pl.when(p == 0)
        def _():
            xb = x_ref[...]
            psum_ref[pl.ds(b * BLK, BLK), :] = jnp.sum(
                xb * xb, axis=1, keepdims=True)
            cache_ref[pl.ds(b * BLK, BLK), :] = xb.astype(jnp.bfloat16)

        @pl.when(jnp.logical_and(p == 1, b == 0))
        def _():
            rdma = pltpu.make_async_remote_copy(
                src_ref=psum_ref,
                dst_ref=peer_ref,
                send_sem=send_sem,
                recv_sem=recv_sem,
                device_id=(my_x, 1 - my_y),
                device_id_type=pl.DeviceIdType.MESH,
            )
            rdma.start()
            rdma.wait()

        @pl.when(p == 1)
        def _():
            rows = pl.ds(b * BLK, BLK)
            total = psum_ref[rows, :] + peer_ref[rows, :]
            scale = lax.rsqrt(total * (1.0 / N_GLOBAL) + EPS)
            xb = cache_ref[rows, :].astype(jnp.float32)
            out_ref[...] = (g_ref[...] * xb * scale).astype(jnp.bfloat16)

    return pl.pallas_call(
        body,
        grid=(2, nb),
        in_specs=[
            pl.BlockSpec((BLK, n), lambda p, b: (b * (1 - p), 0)),
            pl.BlockSpec((1, n), lambda p, b: (0, 0)),
        ],
        out_specs=pl.BlockSpec((BLK, n), lambda p, b: (b * p, 0)),
        out_shape=jax.ShapeDtypeStruct((m, n), jnp.bfloat16),
        scratch_shapes=[
            pltpu.VMEM((m, n), jnp.bfloat16),
            pltpu.VMEM((m, 1), jnp.float32),
            pltpu.VMEM((m, 1), jnp.float32),
            pltpu.SemaphoreType.DMA,
            pltpu.SemaphoreType.DMA,
        ],
        compiler_params=pltpu.CompilerParams(
            dimension_semantics=("arbitrary", "arbitrary"),
        ),
    )(x, gamma2)


# baseline (device time: 88093 ns/iter reference)
import jax
import jax.numpy as jnp
from jax import lax
from jax.experimental import pallas as pl
from jax.experimental.pallas import tpu as pltpu

N_GLOBAL = 4096
EPS = 1e-5
BLK = 512


def kernel(x, gamma):
    m, n = x.shape
    nb = m // BLK
    gamma2 = gamma.reshape(1, n)

    def body(x_ref, g_ref, out_ref, cache_ref, psum_ref, peer_ref,
             send_sem, recv_sem):
        p = pl.program_id(0)
        b = pl.program_id(1)
        my_x = lax.axis_index("x")
        my_y = lax.axis_index("y")

        @pl.when(p == 0)
        def _():
            xb = x_ref[...]
            psum_ref[pl.ds(b * BLK, BLK), :] = jnp.sum(
                xb * xb, axis=1, keepdims=True)
            cache_ref[pl.ds(b * BLK, BLK), :] = xb.astype(jnp.bfloat16)

        @pl.when(jnp.logical_and(p == 1, b == 0))
        def _():
            rdma = pltpu.make_async_remote_copy(
                src_ref=psum_ref,
                dst_ref=peer_ref,
                send_sem=send_sem,
                recv_sem=recv_sem,
                device_id=(my_x, 1 - my_y),
                device_id_type=pl.DeviceIdType.MESH,
            )
            rdma.start()
            rdma.wait()

        @pl.when(p == 1)
        def _():
            rows = pl.ds(b * BLK, BLK)
            total = psum_ref[rows, :] + peer_ref[rows, :]
            scale = lax.rsqrt(total * (1.0 / N_GLOBAL) + EPS)
            xb = cache_ref[rows, :].astype(jnp.float32)
            out_ref[...] = (g_ref[...] * xb * scale).astype(jnp.bfloat16)

    return pl.pallas_call(
        body,
        grid=(2, nb),
        in_specs=[
            pl.BlockSpec((BLK, n), lambda p, b: (b * (1 - p), 0)),
            pl.BlockSpec((1, n), lambda p, b: (0, 0)),
        ],
        out_specs=pl.BlockSpec((BLK, n), lambda p, b: (b * p, 0)),
        out_shape=jax.ShapeDtypeStruct((m, n), jnp.bfloat16),
        scratch_shapes=[
            pltpu.VMEM((m, n), jnp.bfloat16),
            pltpu.VMEM((m, 1), jnp.float32),
            pltpu.VMEM((m, 1), jnp.float32),
            pltpu.SemaphoreType.DMA,
            pltpu.SemaphoreType.DMA,
        ],
        compiler_params=pltpu.CompilerParams(
            dimension_semantics=("arbitrary", "arbitrary"),
            vmem_limit_bytes=56 * 1024 * 1024,
        ),
    )(x, gamma2)


# device time: 55563 ns/iter; 1.5855x vs baseline; 1.5855x over previous
import jax
import jax.numpy as jnp
from jax import lax
from jax.experimental import pallas as pl
from jax.experimental.pallas import tpu as pltpu

N_GLOBAL = 4096
EPS = 1e-5
BLK = 512


def kernel(x, gamma):
    m, n = x.shape
    nb = m // BLK
    gamma2 = gamma.reshape(1, n)

    def body(x_ref, g_ref, out_ref, cache_ref, psum_ref, peer_ref,
             send_sem, recv_sem):
        p = pl.program_id(0)
        b = pl.program_id(1)
        my_x = lax.axis_index("x")
        my_y = lax.axis_index("y")

        @pl.when(p == 0)
        def _():
            xb = x_ref[...]
            s = jnp.sum(xb * xb, axis=1, keepdims=True)
            psum_ref[:, pl.ds(b * BLK, BLK)] = jnp.swapaxes(s, 0, 1)
            cache_ref[pl.ds(b * BLK, BLK), :] = xb.astype(jnp.bfloat16)

        @pl.when(jnp.logical_and(p == 1, b == 0))
        def _():
            rdma = pltpu.make_async_remote_copy(
                src_ref=psum_ref,
                dst_ref=peer_ref,
                send_sem=send_sem,
                recv_sem=recv_sem,
                device_id=(my_x, 1 - my_y),
                device_id_type=pl.DeviceIdType.MESH,
            )
            rdma.start()
            rdma.wait()

        @pl.when(p == 1)
        def _():
            cols = pl.ds(b * BLK, BLK)
            total = psum_ref[:, cols] + peer_ref[:, cols]
            scale_t = lax.rsqrt(total * (1.0 / N_GLOBAL) + EPS)
            scale = jnp.swapaxes(scale_t, 0, 1)
            xb = cache_ref[pl.ds(b * BLK, BLK), :].astype(jnp.float32)
            out_ref[...] = (g_ref[...] * xb * scale).astype(jnp.bfloat16)

    return pl.pallas_call(
        body,
        grid=(2, nb),
        in_specs=[
            pl.BlockSpec((BLK, n), lambda p, b: (b * (1 - p), 0)),
            pl.BlockSpec((1, n), lambda p, b: (0, 0)),
        ],
        out_specs=pl.BlockSpec((BLK, n), lambda p, b: (b * p, 0)),
        out_shape=jax.ShapeDtypeStruct((m, n), jnp.bfloat16),
        scratch_shapes=[
            pltpu.VMEM((m, n), jnp.bfloat16),
            pltpu.VMEM((1, m), jnp.float32),
            pltpu.VMEM((1, m), jnp.float32),
            pltpu.SemaphoreType.DMA,
            pltpu.SemaphoreType.DMA,
        ],
        compiler_params=pltpu.CompilerParams(
            dimension_semantics=("arbitrary", "arbitrary"),
            vmem_limit_bytes=56 * 1024 * 1024,
        ),
    )(x, gamma2)


# device time: 50745 ns/iter; 1.7360x vs baseline; 1.0949x over previous
import jax
import jax.numpy as jnp
from jax import lax
from jax.experimental import pallas as pl
from jax.experimental.pallas import tpu as pltpu

N_GLOBAL = 4096
EPS = 1e-5
BLK = 512


def kernel(x, gamma):
    m, n = x.shape
    nb = m // BLK
    gamma2 = gamma.reshape(1, n)

    def body(x_hbm, g_ref, out_hbm, xbuf, obuf, cache_ref, psum_ref,
             peer_ref, scale_ref, in_sems, out_sems, send_sem, recv_sem):
        my_x = lax.axis_index("x")
        my_y = lax.axis_index("y")

        def fetch(b, slot):
            return pltpu.make_async_copy(
                x_hbm.at[pl.ds(b * BLK, BLK), :], xbuf.at[slot],
                in_sems.at[slot])

        def store(b, slot):
            return pltpu.make_async_copy(
                obuf.at[slot], out_hbm.at[pl.ds(b * BLK, BLK), :],
                out_sems.at[slot])

        bsem = pltpu.get_barrier_semaphore()
        pl.semaphore_signal(bsem, inc=1, device_id=(my_x, 1 - my_y),
                            device_id_type=pl.DeviceIdType.MESH)
        pl.semaphore_wait(bsem, 1)

        fetch(0, 0).start()
        for b in range(nb):
            slot = b % 2
            if b + 1 < nb:
                fetch(b + 1, 1 - slot).start()
            fetch(b, slot).wait()
            xb = xbuf[slot]
            s = jnp.sum(xb * xb, axis=1, keepdims=True)
            psum_ref[:, pl.ds(b * BLK, BLK)] = jnp.swapaxes(s, 0, 1)
            cache_ref[pl.ds(b * BLK, BLK), :] = xb.astype(jnp.bfloat16)

        rdma = pltpu.make_async_remote_copy(
            src_ref=psum_ref, dst_ref=peer_ref,
            send_sem=send_sem, recv_sem=recv_sem,
            device_id=(my_x, 1 - my_y),
            device_id_type=pl.DeviceIdType.MESH)
        rdma.start()
        rdma.wait()

        scale_ref[...] = lax.rsqrt(
            (psum_ref[...] + peer_ref[...]) * (1.0 / N_GLOBAL) + EPS)

        for b in range(nb):
            slot = b % 2
            if b >= 2:
                store(b - 2, slot).wait()
            sc = jnp.swapaxes(scale_ref[:, pl.ds(b * BLK, BLK)], 0, 1)
            xb = cache_ref[pl.ds(b * BLK, BLK), :].astype(jnp.float32)
            obuf[slot] = (g_ref[...] * xb * sc).astype(jnp.bfloat16)
            store(b, slot).start()
        store(nb - 2, (nb - 2) % 2).wait()
        store(nb - 1, (nb - 1) % 2).wait()

    return pl.pallas_call(
        body,
        in_specs=[
            pl.BlockSpec(memory_space=pl.ANY),
            pl.BlockSpec(memory_space=pltpu.VMEM),
        ],
        out_specs=pl.BlockSpec(memory_space=pl.ANY),
        out_shape=jax.ShapeDtypeStruct((m, n), jnp.bfloat16),
        scratch_shapes=[
            pltpu.VMEM((2, BLK, n), jnp.float32),
            pltpu.VMEM((2, BLK, n), jnp.bfloat16),
            pltpu.VMEM((m, n), jnp.bfloat16),
            pltpu.VMEM((1, m), jnp.float32),
            pltpu.VMEM((1, m), jnp.float32),
            pltpu.VMEM((1, m), jnp.float32),
            pltpu.SemaphoreType.DMA((2,)),
            pltpu.SemaphoreType.DMA((2,)),
            pltpu.SemaphoreType.DMA,
            pltpu.SemaphoreType.DMA,
        ],
        compiler_params=pltpu.CompilerParams(
            collective_id=0,
            vmem_limit_bytes=56 * 1024 * 1024,
        ),
    )(x, gamma2)


# device time: 50648 ns/iter; 1.7393x vs baseline; 1.0019x over previous
import jax
import jax.numpy as jnp
from jax import lax
from jax.experimental import pallas as pl
from jax.experimental.pallas import tpu as pltpu

N_GLOBAL = 4096
EPS = 1e-5
BLK = 512


def kernel(x, gamma):
    m, n = x.shape
    nb = m // BLK
    gamma2 = gamma.reshape(1, n)

    def body(x_hbm, g_ref, out_hbm, xbuf, obuf, cache_ref, psum_ref,
             peer_ref, scale_ref, in_sems, out_sems, send_sem, recv_sem):
        my_x = lax.axis_index("x")
        my_y = lax.axis_index("y")

        def fetch(b, slot):
            return pltpu.make_async_copy(
                x_hbm.at[pl.ds(b * BLK, BLK), :], xbuf.at[slot],
                in_sems.at[slot])

        def store(b, slot):
            return pltpu.make_async_copy(
                obuf.at[slot], out_hbm.at[pl.ds(b * BLK, BLK), :],
                out_sems.at[slot])

        fetch(0, 0).start()
        for b in range(nb):
            slot = b % 2
            if b + 1 < nb:
                fetch(b + 1, 1 - slot).start()
            fetch(b, slot).wait()
            xb = xbuf[slot]
            s = jnp.sum(xb * xb, axis=1, keepdims=True)
            psum_ref[:, pl.ds(b * BLK, BLK)] = jnp.swapaxes(s, 0, 1)
            cache_ref[pl.ds(b * BLK, BLK), :] = xb.astype(jnp.bfloat16)

        bsem = pltpu.get_barrier_semaphore()
        pl.semaphore_signal(bsem, inc=1, device_id=(my_x, 1 - my_y),
                            device_id_type=pl.DeviceIdType.MESH)
        pl.semaphore_wait(bsem, 1)

        rdma = pltpu.make_async_remote_copy(
            src_ref=psum_ref, dst_ref=peer_ref,
            send_sem=send_sem, recv_sem=recv_sem,
            device_id=(my_x, 1 - my_y),
            device_id_type=pl.DeviceIdType.MESH)
        rdma.start()
        rdma.wait()

        scale_ref[...] = lax.rsqrt(
            (psum_ref[...] + peer_ref[...]) * (1.0 / N_GLOBAL) + EPS)

        for b in range(nb):
            slot = b % 2
            if b >= 2:
                store(b - 2, slot).wait()
            sc = jnp.swapaxes(scale_ref[:, pl.ds(b * BLK, BLK)], 0, 1)
            xb = cache_ref[pl.ds(b * BLK, BLK), :].astype(jnp.float32)
            obuf[slot] = (g_ref[...] * xb * sc).astype(jnp.bfloat16)
            store(b, slot).start()
        store(nb - 2, (nb - 2) % 2).wait()
        store(nb - 1, (nb - 1) % 2).wait()

    return pl.pallas_call(
        body,
        in_specs=[
            pl.BlockSpec(memory_space=pl.ANY),
            pl.BlockSpec(memory_space=pltpu.VMEM),
        ],
        out_specs=pl.BlockSpec(memory_space=pl.ANY),
        out_shape=jax.ShapeDtypeStruct((m, n), jnp.bfloat16),
        scratch_shapes=[
            pltpu.VMEM((2, BLK, n), jnp.float32),
            pltpu.VMEM((2, BLK, n), jnp.bfloat16),
            pltpu.VMEM((m, n), jnp.bfloat16),
            pltpu.VMEM((1, m), jnp.float32),
            pltpu.VMEM((1, m), jnp.float32),
            pltpu.VMEM((1, m), jnp.float32),
            pltpu.SemaphoreType.DMA((2,)),
            pltpu.SemaphoreType.DMA((2,)),
            pltpu.SemaphoreType.DMA,
            pltpu.SemaphoreType.DMA,
        ],
        compiler_params=pltpu.CompilerParams(
            collective_id=0,
            vmem_limit_bytes=56 * 1024 * 1024,
        ),
    )(x, gamma2)


# device time: 47402 ns/iter; 1.8584x vs baseline; 1.0685x over previous
import jax
import jax.numpy as jnp
from jax import lax
from jax.experimental import pallas as pl
from jax.experimental.pallas import tpu as pltpu

N_GLOBAL = 4096
EPS = 1e-5
BLK = 512
NBUF_IN = 4
NBUF_OUT = 3


def kernel(x, gamma):
    m, n = x.shape
    nb = m // BLK
    gamma2 = gamma.reshape(1, n)

    def body(x_hbm, g_ref, out_hbm, xbuf, obuf, cache_ref, psum_ref,
             peer_ref, scale_ref, in_sems, out_sems, send_sem, recv_sem):
        my_x = lax.axis_index("x")
        my_y = lax.axis_index("y")

        def fetch(b, slot):
            return pltpu.make_async_copy(
                x_hbm.at[pl.ds(b * BLK, BLK), :], xbuf.at[slot],
                in_sems.at[slot])

        def store(b, slot):
            return pltpu.make_async_copy(
                obuf.at[slot], out_hbm.at[pl.ds(b * BLK, BLK), :],
                out_sems.at[slot])

        for k in range(NBUF_IN - 1):
            fetch(k, k).start()
        for b in range(nb):
            slot = b % NBUF_IN
            if b + NBUF_IN - 1 < nb:
                fetch(b + NBUF_IN - 1, (b + NBUF_IN - 1) % NBUF_IN).start()
            fetch(b, slot).wait()
            xb = xbuf[slot]
            s = jnp.sum(xb * xb, axis=1, keepdims=True)
            psum_ref[:, pl.ds(b * BLK, BLK)] = jnp.swapaxes(s, 0, 1)
            cache_ref[pl.ds(b * BLK, BLK), :] = xb.astype(jnp.bfloat16)

        bsem = pltpu.get_barrier_semaphore()
        pl.semaphore_signal(bsem, inc=1, device_id=(my_x, 1 - my_y),
                            device_id_type=pl.DeviceIdType.MESH)
        pl.semaphore_wait(bsem, 1)

        rdma = pltpu.make_async_remote_copy(
            src_ref=psum_ref, dst_ref=peer_ref,
            send_sem=send_sem, recv_sem=recv_sem,
            device_id=(my_x, 1 - my_y),
            device_id_type=pl.DeviceIdType.MESH)
        rdma.start()
        rdma.wait()

        scale_ref[...] = lax.rsqrt(
            (psum_ref[...] + peer_ref[...]) * (1.0 / N_GLOBAL) + EPS)

        for b in range(nb):
            slot = b % NBUF_OUT
            if b >= NBUF_OUT:
                store(b - NBUF_OUT, slot).wait()
            sc = jnp.swapaxes(scale_ref[:, pl.ds(b * BLK, BLK)], 0, 1)
            xb = cache_ref[pl.ds(b * BLK, BLK), :].astype(jnp.float32)
            obuf[slot] = (g_ref[...] * xb * sc).astype(jnp.bfloat16)
            store(b, slot).start()
        for b in range(nb - NBUF_OUT, nb):
            store(b, b % NBUF_OUT).wait()

    return pl.pallas_call(
        body,
        in_specs=[
            pl.BlockSpec(memory_space=pl.ANY),
            pl.BlockSpec(memory_space=pltpu.VMEM),
        ],
        out_specs=pl.BlockSpec(memory_space=pl.ANY),
        out_shape=jax.ShapeDtypeStruct((m, n), jnp.bfloat16),
        scratch_shapes=[
            pltpu.VMEM((NBUF_IN, BLK, n), jnp.float32),
            pltpu.VMEM((NBUF_OUT, BLK, n), jnp.bfloat16),
            pltpu.VMEM((m, n), jnp.bfloat16),
            pltpu.VMEM((1, m), jnp.float32),
            pltpu.VMEM((1, m), jnp.float32),
            pltpu.VMEM((1, m), jnp.float32),
            pltpu.SemaphoreType.DMA((NBUF_IN,)),
            pltpu.SemaphoreType.DMA((NBUF_OUT,)),
            pltpu.SemaphoreType.DMA,
            pltpu.SemaphoreType.DMA,
        ],
        compiler_params=pltpu.CompilerParams(
            collective_id=0,
            vmem_limit_bytes=56 * 1024 * 1024,
        ),
    )(x, gamma2)
